# Initial kernel scaffold; baseline (speedup 1.0000x reference)
import jax
import jax.numpy as jnp
from jax.experimental import pallas as pl


def kernel(X, table):
    raise NotImplementedError("write your pallas kernel here")


# speedup vs baseline: 1.0948x; 1.0948x over previous
import functools

import jax
import jax.numpy as jnp
from jax import lax
from jax.experimental import pallas as pl
from jax.experimental.pallas import tpu as pltpu
from jax.experimental.pallas import tpu_sc as plsc

_D = 32
_NC = 2
_NS = 16
_NW = _NC * _NS
_CHUNK = 1024


@functools.lru_cache(maxsize=None)
def _build(b_total: int):
    b_per_w = b_total // _NW
    n_chunk = b_per_w // _CHUNK
    mesh = plsc.VectorSubcoreMesh(core_axis_name="c", subcore_axis_name="s")

    @functools.partial(
        pl.kernel,
        mesh=mesh,
        out_type=jax.ShapeDtypeStruct((b_total, _D), jnp.float32),
        scratch_types=[
            pltpu.VMEM((_CHUNK,), jnp.int32),
            pltpu.VMEM((_CHUNK, _D), jnp.float32),
            pltpu.SemaphoreType.DMA,
        ],
        compiler_params=pltpu.CompilerParams(use_tc_tiling_on_sc=False),
    )
    def gather(idx_hbm, table_hbm, out_hbm, idx_v, rows_v, sem):
        wid = lax.axis_index("s") * _NC + lax.axis_index("c")
        base = wid * b_per_w

        def step(i, carry):
            off = base + i * _CHUNK
            pltpu.sync_copy(idx_hbm.at[pl.ds(off, _CHUNK)], idx_v)
            pltpu.async_copy(table_hbm.at[idx_v], rows_v, sem).wait()
            pltpu.sync_copy(rows_v, out_hbm.at[pl.ds(off, _CHUNK)])
            return carry

        lax.fori_loop(0, n_chunk, step, 0)

    return gather


def kernel(X, table):
    B, H = X.shape
    idx = X.reshape(B * H).astype(jnp.int32)
    out = _build(B * H)(idx, table)
    return out.reshape(B, H, _D)


# speedup vs baseline: 1.1112x; 1.0150x over previous
import functools

import jax
import jax.numpy as jnp
from jax import lax
from jax.experimental import pallas as pl
from jax.experimental.pallas import tpu as pltpu
from jax.experimental.pallas import tpu_sc as plsc

_D = 32
_NC = 2
_NS = 16
_NW = _NC * _NS
_CHUNK = 1280


@functools.lru_cache(maxsize=None)
def _build(b_total: int):
    b_per_w = b_total // _NW
    n_chunk = b_per_w // _CHUNK
    n2 = n_chunk // 2
    assert n_chunk % 2 == 0 and n2 >= 3
    mesh = plsc.VectorSubcoreMesh(core_axis_name="c", subcore_axis_name="s")

    @functools.partial(
        pl.kernel,
        mesh=mesh,
        out_type=jax.ShapeDtypeStruct((b_total, _D), jnp.float32),
        scratch_types=[
            pltpu.VMEM((b_per_w,), jnp.int32),
            pltpu.VMEM((2, _CHUNK, _D), jnp.float32),
            pltpu.SemaphoreType.DMA,
            pltpu.SemaphoreType.DMA,
            pltpu.SemaphoreType.DMA,
            pltpu.SemaphoreType.DMA,
        ],
        compiler_params=pltpu.CompilerParams(use_tc_tiling_on_sc=False),
    )
    def gather(idx_hbm, table_hbm, out_hbm, idx_v, rows_v, g0, g1, w0, w1):
        wid = lax.axis_index("s") * _NC + lax.axis_index("c")
        base = wid * b_per_w
        gsem = (g0, g1)
        wsem = (w0, w1)

        def g_copy(i, b):
            return pltpu.make_async_copy(
                table_hbm.at[idx_v.at[pl.ds(i * _CHUNK, _CHUNK)]],
                rows_v.at[b], gsem[b])

        def w_copy(i, b):
            return pltpu.make_async_copy(
                rows_v.at[b], out_hbm.at[pl.ds(base + i * _CHUNK, _CHUNK)],
                wsem[b])

        pltpu.sync_copy(idx_hbm.at[pl.ds(base, b_per_w)], idx_v)

        g_copy(0, 0).start()
        g_copy(0, 0).wait()
        w_copy(0, 0).start()
        g_copy(1, 1).start()
        g_copy(1, 1).wait()
        w_copy(1, 1).start()
        w_copy(0, 0).wait()
        g_copy(2, 0).start()

        def body(j, carry):
            i = 2 * j
            g_copy(i, 0).wait()
            w_copy(i, 0).start()
            w_copy(i - 1, 1).wait()
            g_copy(i + 1, 1).start()
            g_copy(i + 1, 1).wait()
            w_copy(i + 1, 1).start()
            w_copy(i, 0).wait()
            g_copy(i + 2, 0).start()
            return carry

        lax.fori_loop(1, n2 - 1, body, 0)

        i = n_chunk - 2
        g_copy(i, 0).wait()
        w_copy(i, 0).start()
        w_copy(i - 1, 1).wait()
        g_copy(i + 1, 1).start()
        g_copy(i + 1, 1).wait()
        w_copy(i + 1, 1).start()
        w_copy(i, 0).wait()
        w_copy(i + 1, 1).wait()

    return gather


def kernel(X, table):
    B, H = X.shape
    idx = X.reshape(B * H).astype(jnp.int32)
    out = _build(B * H)(idx, table)
    return out.reshape(B, H, _D)


# speedup vs baseline: 1.1139x; 1.0024x over previous
import functools

import jax
import jax.numpy as jnp
from jax import lax
from jax.experimental import pallas as pl
from jax.experimental.pallas import tpu as pltpu
from jax.experimental.pallas import tpu_sc as plsc

_D = 32
_NC = 2
_NS = 16
_NW = _NC * _NS
_CHUNK = 512
_NBUF = 5


@functools.lru_cache(maxsize=None)
def _build(b_total: int):
    b_per_w = b_total // _NW
    n_chunk = b_per_w // _CHUNK
    n_grp = n_chunk // _NBUF
    assert n_chunk % _NBUF == 0 and n_grp >= 3
    mesh = plsc.VectorSubcoreMesh(core_axis_name="c", subcore_axis_name="s")

    @functools.partial(
        pl.kernel,
        mesh=mesh,
        out_type=jax.ShapeDtypeStruct((b_total, _D), jnp.float32),
        scratch_types=[
            pltpu.VMEM((b_per_w,), jnp.int32),
            pltpu.VMEM((_NBUF, _CHUNK, _D), jnp.float32),
            [pltpu.SemaphoreType.DMA] * _NBUF,
            [pltpu.SemaphoreType.DMA] * _NBUF,
        ],
        compiler_params=pltpu.CompilerParams(use_tc_tiling_on_sc=False),
    )
    def gather(idx_hbm, table_hbm, out_hbm, idx_v, rows_v, gsem, wsem):
        wid = lax.axis_index("s") * _NC + lax.axis_index("c")
        base = wid * b_per_w

        def g_copy(i, b):
            return pltpu.make_async_copy(
                table_hbm.at[idx_v.at[pl.ds(i * _CHUNK, _CHUNK)]],
                rows_v.at[b], gsem[b])

        def w_copy(i, b):
            return pltpu.make_async_copy(
                rows_v.at[b], out_hbm.at[pl.ds(base + i * _CHUNK, _CHUNK)],
                wsem[b])

        pltpu.sync_copy(idx_hbm.at[pl.ds(base, b_per_w)], idx_v)

        for b in range(_NBUF):
            g_copy(b, b).start()

        def slots(i0, launch_next):
            for b in range(_NBUF):
                i = i0 + b
                g_copy(i, b).wait()
                w_copy(i, b).start()
                w_copy(i, b).wait()
                if launch_next:
                    g_copy(i + _NBUF, b).start()

        def body(j, carry):
            slots(j * _NBUF, True)
            return carry

        lax.fori_loop(0, n_grp - 1, body, 0)
        slots((n_grp - 1) * _NBUF, False)

    return gather


def kernel(X, table):
    B, H = X.shape
    idx = X.reshape(B * H).astype(jnp.int32)
    out = _build(B * H)(idx, table)
    return out.reshape(B, H, _D)


# speedup vs baseline: 1.8052x; 1.6206x over previous
import functools

import jax
import jax.numpy as jnp
from jax import lax
from jax.experimental import pallas as pl
from jax.experimental.pallas import tpu as pltpu
from jax.experimental.pallas import tpu_sc as plsc

_D = 32
_NC = 2
_NS = 16
_NW = _NC * _NS
_S = 8
_NBUF = 4


@functools.lru_cache(maxsize=None)
def _build(batch: int, hist: int):
    s_per_w = batch // _NW
    n_chunk = s_per_w // _S
    n_grp = n_chunk // _NBUF
    assert n_chunk % _NBUF == 0 and n_grp >= 3
    mesh = plsc.VectorSubcoreMesh(core_axis_name="c", subcore_axis_name="s")

    @functools.partial(
        pl.kernel,
        mesh=mesh,
        out_type=jax.ShapeDtypeStruct((batch, hist, _D), jnp.float32),
        scratch_types=[
            pltpu.VMEM((s_per_w, hist), jnp.int32),
            pltpu.VMEM((_NBUF, _S, hist, _D), jnp.float32),
            [pltpu.SemaphoreType.DMA] * _NBUF,
            [pltpu.SemaphoreType.DMA] * _NBUF,
        ],
        compiler_params=pltpu.CompilerParams(use_tc_tiling_on_sc=False),
    )
    def gather(x_hbm, table_hbm, out_hbm, idx_v, rows_v, gsem, wsem):
        wid = lax.axis_index("s") * _NC + lax.axis_index("c")
        base = wid * s_per_w

        def g_copy_one(i, b, k):
            return pltpu.make_async_copy(
                table_hbm.at[idx_v.at[i * _S + k, :]],
                rows_v.at[b].at[k], gsem[b])

        def g_start(i, b):
            for k in range(_S):
                g_copy_one(i, b, k).start()

        def g_wait(i, b):
            for k in range(_S):
                g_copy_one(i, b, k).wait()

        def w_copy(i, b):
            return pltpu.make_async_copy(
                rows_v.at[b], out_hbm.at[pl.ds(base + i * _S, _S), :, :],
                wsem[b])

        pltpu.sync_copy(x_hbm.at[pl.ds(base, s_per_w), :], idx_v)

        for b in range(_NBUF):
            g_start(b, b)

        def slots(i0, launch_next):
            for b in range(_NBUF):
                i = i0 + b
                g_wait(i, b)
                w_copy(i, b).start()
                w_copy(i, b).wait()
                if launch_next:
                    g_start(i + _NBUF, b)

        def body(j, carry):
            slots(j * _NBUF, True)
            return carry

        lax.fori_loop(0, n_grp - 1, body, 0)
        slots((n_grp - 1) * _NBUF, False)

    return gather


def kernel(X, table):
    B, H = X.shape
    return _build(B, H)(X.astype(jnp.int32), table)
